# baseline (speedup 1.0000x reference)
import functools

import jax
import jax.numpy as jnp
from jax import lax
from jax.experimental import pallas as pl
from jax.experimental.pallas import tpu as pltpu
from jax.experimental.pallas import tpu_sc as plsc

_NUM_NODES = 1000000
_EMBED = 64
_BATCH = 16384

_INFO = plsc.get_sparse_core_info()
_NC = _INFO.num_cores
_NS = _INFO.num_subcores
_NW = _NC * _NS
_L = 16

_CH = 256
_CH_SHIFT = 8
_NFULL = (_NUM_NODES // _CH)
_TAIL_BASE = _NFULL * _CH
_TAIL_LEN = _NUM_NODES - _TAIL_BASE
_NBUF = 4
_GFULL = _NFULL // _NW
_NEXTRA = _NFULL - _GFULL * _NW

_SLAB_BYTES = _EMBED * _CH * 4
_ROW_BYTES = _EMBED * 4
_IOTA = None


def _gather_body(idx_hbm, tablet_hbm, tail_hbm, out_hbm, idx_v, loc_v, hit_v,
                 slab_v, tail_v, ring_v, sem_slab, sem_row):
    wid = lax.axis_index("s") * _NC + lax.axis_index("c")
    n_g = _GFULL + (wid < _NEXTRA).astype(jnp.int32)
    iota = lax.iota(jnp.int32, _L)

    def slab_fetch(g_chunk, buf):
        cb = pl.multiple_of((wid + g_chunk * _NW) * _CH, 128)
        for t in range(_EMBED // 8):
            pltpu.async_copy(
                tablet_hbm.at[pl.ds(8 * t, 8), pl.ds(cb, _CH)],
                slab_v.at[buf, pl.ds(8 * t, 8), :], sem_slab)

    def slab_wait(src_ref, dst_ref):
        pltpu.make_async_copy(src_ref, dst_ref, sem_slab).wait()

    for b in range(_NBUF):
        slab_fetch(jnp.minimum(b, n_g - 1), b)
    pltpu.sync_copy(idx_hbm.at[pl.ds(0, _BATCH)], idx_v)

    def filt(i, n):
        lv = idx_v[pl.ds(i * _L, _L)]
        slots = iota + i * _L
        cid = lv >> _CH_SHIFT
        off = lv & (_CH - 1)
        mine = (cid & (_NW - 1)) == wid
        pack = slots | (off << 14) | ((cid >> 5) << 23)
        plsc.store_compressed(loc_v.at[pl.ds(n, _L)], pack, mask=mine)
        return n + plsc.all_reduce_population_count(mine)[0]

    nloc = lax.fori_loop(0, _BATCH // _L, filt, 0)
    nblk = (nloc + _L - 1) >> 4

    def process_chunk(g_match, gather_row):
        def cfilt(j, nh):
            pv = loc_v[pl.ds(j * _L, _L)]
            valid = (iota + j * _L) < nloc
            m = valid & ((pv >> 23) == g_match)
            plsc.store_compressed(hit_v.at[pl.ds(nh, _L)], pv, mask=m)
            return nh + plsc.all_reduce_population_count(m)[0]

        nh = lax.fori_loop(0, nblk, cfilt, 0)
        h0 = hit_v[pl.ds(0, _L)][0]
        hit_v[pl.ds(nh, _L)] = jnp.full((_L,), h0, jnp.int32)

        def hit_block(b, carry):
            pv = hit_v[pl.ds(b * _L, _L)]
            offs = (pv >> 14) & 511
            slots = pv & (_BATCH - 1)
            for lane in range(_L):
                off = offs[lane]
                slot = slots[lane]
                ring = lane
                offv = jnp.full((_L,), off, jnp.int32)
                for k in range(_EMBED // _L):
                    v = gather_row(iota + k * _L, offv)
                    ring_v[ring, pl.ds(k * _L, _L)] = v
                pltpu.async_copy(ring_v.at[pl.ds(ring, 1), :],
                                 out_hbm.at[pl.ds(slot, 1), :], sem_row)
            pltpu.make_async_copy(out_hbm.at[pl.ds(0, _L), :],
                                  ring_v.at[pl.ds(0, _L), :], sem_row).wait()
            return carry

        lax.fori_loop(0, (nh + _L - 1) >> 4, hit_block, 0)

    def chunk_body(g, buf):
        slab_wait(tablet_hbm.at[:, pl.ds(0, _CH)], slab_v.at[0])
        bufv = jnp.full((_L,), buf, jnp.int32)
        process_chunk(g, lambda cv, ov: plsc.load_gather(slab_v, [bufv, cv, ov]))
        slab_fetch(jnp.minimum(g + _NBUF, n_g - 1), buf)
        return jnp.where(buf == _NBUF - 1, 0, buf + 1)

    lax.fori_loop(0, n_g, chunk_body, 0)
    for _ in range(_NBUF):
        slab_wait(tablet_hbm.at[:, pl.ds(0, _CH)], slab_v.at[0])

    pltpu.async_copy(tail_hbm, tail_v, sem_slab)
    slab_wait(tail_hbm, tail_v)
    process_chunk(_GFULL + (wid < _NEXTRA).astype(jnp.int32),
                  lambda cv, ov: plsc.load_gather(tail_v, [cv, ov]))


@jax.jit
def _embed_lookup(node_ids, node_embed_weight):
    run = pl.kernel(
        _gather_body,
        out_type=jax.ShapeDtypeStruct((_BATCH, _EMBED), jnp.float32),
        mesh=plsc.VectorSubcoreMesh(core_axis_name="c", subcore_axis_name="s"),
        scratch_types=[
            pltpu.VMEM((_BATCH,), jnp.int32),
            pltpu.VMEM((_BATCH + _L,), jnp.int32),
            pltpu.VMEM((_BATCH + _L,), jnp.int32),
            pltpu.VMEM((_NBUF, _EMBED, _CH), jnp.float32),
            pltpu.VMEM((_EMBED, _TAIL_LEN), jnp.float32),
            pltpu.VMEM((_L, _EMBED), jnp.float32),
            pltpu.SemaphoreType.DMA,
            pltpu.SemaphoreType.DMA,
        ],
        compiler_params=pltpu.CompilerParams(needs_layout_passes=False),
    )
    tablet = node_embed_weight.T
    return run(node_ids, tablet, tablet[:, _TAIL_BASE:])


def kernel(node_ids, node_embed_weight):
    return _embed_lookup(node_ids.astype(jnp.int32), node_embed_weight)


# speedup vs baseline: 1.2983x; 1.2983x over previous
import functools

import jax
import jax.numpy as jnp
from jax import lax
from jax.experimental import pallas as pl
from jax.experimental.pallas import tpu as pltpu
from jax.experimental.pallas import tpu_sc as plsc

_NUM_NODES = 1000000
_EMBED = 64
_BATCH = 16384

_INFO = plsc.get_sparse_core_info()
_NC = _INFO.num_cores
_NS = _INFO.num_subcores
_NW = _NC * _NS
_L = 16

_CH = 512
_NFULL = (_NUM_NODES // _CH)
_TAIL_BASE = _NFULL * _CH
_TAIL_LEN = _NUM_NODES - _TAIL_BASE
_G_TAIL = 63

_SLAB_BYTES = _EMBED * _CH * 4
_ROW_BYTES = _EMBED * 4
_IOTA = None


def _gather_body(idx_hbm, tablet_hbm, tail_hbm, out_hbm, idx_v, loc_v, hit_v,
                 slab_v, tail_v, ring_v, sem_slab, sem_row):
    wid = lax.axis_index("s") * _NC + lax.axis_index("c")
    n_g = 61 + (wid == 0).astype(jnp.int32)
    iota = lax.iota(jnp.int32, _L)

    def slab_fetch(g_chunk, buf):
        cb = pl.multiple_of((wid + g_chunk * _NW) * _CH, _CH)
        pltpu.async_copy(tablet_hbm.at[:, pl.ds(cb, _CH)], slab_v.at[buf],
                         sem_slab)

    def slab_wait(src_ref, dst_ref):
        pltpu.make_async_copy(src_ref, dst_ref, sem_slab).wait()

    slab_fetch(0, 0)
    slab_fetch(jnp.minimum(1, n_g - 1), 1)
    pltpu.sync_copy(idx_hbm.at[pl.ds(0, _BATCH)], idx_v)

    def filt(i, n):
        lv = idx_v[pl.ds(i * _L, _L)]
        slots = iota + i * _L
        cid = lv >> 9
        mine = (cid & (_NW - 1)) == wid
        pack = slots | ((lv & (_CH - 1)) << 14) | ((cid >> 5) << 23)
        plsc.store_compressed(loc_v.at[pl.ds(n, _L)], pack, mask=mine)
        return n + plsc.all_reduce_population_count(mine)[0]

    nloc = lax.fori_loop(0, _BATCH // _L, filt, 0)
    nblk = (nloc + _L - 1) >> 4

    def process_chunk(g_match, gather_row):
        def cfilt(j, nh):
            pv = loc_v[pl.ds(j * _L, _L)]
            valid = (iota + j * _L) < nloc
            m = valid & ((pv >> 23) == g_match)
            plsc.store_compressed(hit_v.at[pl.ds(nh, _L)], pv, mask=m)
            return nh + plsc.all_reduce_population_count(m)[0]

        nh = lax.fori_loop(0, nblk, cfilt, 0)
        h0 = hit_v[pl.ds(0, _L)][0]
        hit_v[pl.ds(nh, _L)] = jnp.full((_L,), h0, jnp.int32)

        def hit_block(b, carry):
            pv = hit_v[pl.ds(b * _L, _L)]
            offs = (pv >> 14) & (_CH - 1)
            slots = pv & (_BATCH - 1)
            par = (b & 1) * _L
            for lane in range(_L):
                off = offs[lane]
                slot = slots[lane]
                ring = par + lane
                offv = jnp.full((_L,), off, jnp.int32)
                for k in range(_EMBED // _L):
                    v = gather_row(iota + k * _L, offv)
                    ring_v[ring, pl.ds(k * _L, _L)] = v
                pltpu.async_copy(ring_v.at[pl.ds(ring, 1), :],
                                 out_hbm.at[pl.ds(slot, 1), :], sem_row)
            pltpu.make_async_copy(out_hbm.at[pl.ds(0, _L), :],
                                  ring_v.at[pl.ds(0, _L), :], sem_row).wait()
            return carry

        lax.fori_loop(0, (nh + _L - 1) >> 4, hit_block, 0)

    def chunk_body(g, carry):
        buf = g & 1
        slab_wait(tablet_hbm.at[:, pl.ds(0, _CH)], slab_v.at[0])
        bufv = jnp.full((_L,), buf, jnp.int32)
        process_chunk(g, lambda cv, ov: plsc.load_gather(slab_v, [bufv, cv, ov]))
        slab_fetch(jnp.minimum(g + 2, n_g - 1), buf)
        return carry

    lax.fori_loop(0, n_g, chunk_body, 0)
    slab_wait(tablet_hbm.at[:, pl.ds(0, _CH)], slab_v.at[0])
    slab_wait(tablet_hbm.at[:, pl.ds(0, _CH)], slab_v.at[0])

    pltpu.async_copy(tail_hbm, tail_v, sem_slab)
    slab_wait(tail_hbm, tail_v)
    process_chunk(61 + (wid == 0).astype(jnp.int32),
                  lambda cv, ov: plsc.load_gather(tail_v, [cv, ov]))


@jax.jit
def _embed_lookup(node_ids, node_embed_weight):
    run = pl.kernel(
        _gather_body,
        out_type=jax.ShapeDtypeStruct((_BATCH, _EMBED), jnp.float32),
        mesh=plsc.VectorSubcoreMesh(core_axis_name="c", subcore_axis_name="s"),
        scratch_types=[
            pltpu.VMEM((_BATCH,), jnp.int32),
            pltpu.VMEM((_BATCH + _L,), jnp.int32),
            pltpu.VMEM((_BATCH + _L,), jnp.int32),
            pltpu.VMEM((2, _EMBED, _CH), jnp.float32),
            pltpu.VMEM((_EMBED, _TAIL_LEN), jnp.float32),
            pltpu.VMEM((2 * _L, _EMBED), jnp.float32),
            pltpu.SemaphoreType.DMA,
            pltpu.SemaphoreType.DMA,
        ],
        compiler_params=pltpu.CompilerParams(needs_layout_passes=False),
    )
    tablet = node_embed_weight.T
    return run(node_ids, tablet, tablet[:, _TAIL_BASE:])


def kernel(node_ids, node_embed_weight):
    return _embed_lookup(node_ids.astype(jnp.int32), node_embed_weight)


# speedup vs baseline: 1.3235x; 1.0194x over previous
import functools

import jax
import jax.numpy as jnp
from jax import lax
from jax.experimental import pallas as pl
from jax.experimental.pallas import tpu as pltpu
from jax.experimental.pallas import tpu_sc as plsc

_NUM_NODES = 1000000
_EMBED = 64
_BATCH = 16384

_INFO = plsc.get_sparse_core_info()
_NC = _INFO.num_cores
_NS = _INFO.num_subcores
_NW = _NC * _NS
_L = 16

_CH = 512
_GPC = _CH // 128
_NFULL = _NUM_NODES // _CH
_TAIL_BASE = _NFULL * _CH
_TAIL_LEN = _NUM_NODES - _TAIL_BASE
_GFULL = _NFULL // _NW
_NEXTRA = _NFULL - _GFULL * _NW
_CNT_CAP = (_GFULL + 1) * _GPC + _L


def _gather_body(idx_hbm, tablet_hbm, tail_hbm, out_hbm, idx_v, loc_v, hit_v,
                 slab_v, tail_v, ring_v, cnt_v, gl_v, sem_slab, sem_row):
    wid = lax.axis_index("s") * _NC + lax.axis_index("c")
    n_g = _GFULL + (wid < _NEXTRA).astype(jnp.int32)
    iota = lax.iota(jnp.int32, _L)
    ones = jnp.full((_L,), 1, jnp.int32)

    def fetch_full(g_chunk, buf):
        cb = pl.multiple_of((wid + g_chunk * _NW) * _CH, _CH)
        pltpu.async_copy(tablet_hbm.at[:, pl.ds(cb, _CH)], slab_v.at[buf],
                         sem_slab)

    fetch_full(0, 0)
    fetch_full(jnp.minimum(1, n_g - 1), 1)
    pltpu.sync_copy(idx_hbm.at[pl.ds(0, _BATCH)], idx_v)

    def zero_cnt(i, c):
        cnt_v[pl.ds(i * _L, _L)] = jnp.zeros((_L,), jnp.int32)
        return c

    lax.fori_loop(0, _CNT_CAP // _L, zero_cnt, 0)

    def filt(i, n):
        lv = idx_v[pl.ds(i * _L, _L)]
        slots = iota + i * _L
        cid = lv >> 9
        off = lv & (_CH - 1)
        mine = (cid & (_NW - 1)) == wid
        gslot = cid >> 5
        pack = slots | (off << 14) | (gslot << 23)
        plsc.store_compressed(loc_v.at[pl.ds(n, _L)], pack, mask=mine)
        plsc.addupdate_scatter(cnt_v, [(gslot << 2) | (off >> 7)], ones,
                               mask=mine)
        return n + plsc.all_reduce_population_count(mine)[0]

    nloc = lax.fori_loop(0, _BATCH // _L, filt, 0)
    nblk = (nloc + _L - 1) >> 4

    def group_count(g_chunk):
        cv = cnt_v[pl.ds(g_chunk * _GPC, _L)]
        m = (cv > 0) & (iota < _GPC)
        return m, plsc.all_reduce_population_count(m)[0]

    def fetch_skip(g_chunk, buf):
        m, k = group_count(g_chunk)
        plsc.store_compressed(gl_v.at[pl.ds(0, _L)], iota, mask=m)
        cb = (wid + g_chunk * _NW) * _CH

        def one(j, c):
            gi = plsc.load_gather(gl_v, [jnp.full((_L,), j, jnp.int32)])[0]
            src = pl.multiple_of(cb + gi * 128, 128)
            dst = pl.multiple_of(j * 128, 128)
            pltpu.async_copy(tablet_hbm.at[:, pl.ds(src, 128)],
                             slab_v.at[buf, :, pl.ds(dst, 128)], sem_slab)
            return c

        lax.fori_loop(0, k, one, 0)

    def wait_units(k):
        def one(j, c):
            pltpu.make_async_copy(tablet_hbm.at[:, pl.ds(0, 128)],
                                  slab_v.at[0, :, pl.ds(0, 128)],
                                  sem_slab).wait()
            return c

        lax.fori_loop(0, k, one, 0)

    def process_chunk(g_match, gather_row, occ):
        def cfilt(j, nh):
            pv = loc_v[pl.ds(j * _L, _L)]
            valid = (iota + j * _L) < nloc
            m = valid & ((pv >> 23) == g_match)
            plsc.store_compressed(hit_v.at[pl.ds(nh, _L)], pv, mask=m)
            return nh + plsc.all_reduce_population_count(m)[0]

        nh = lax.fori_loop(0, nblk, cfilt, 0)
        h0 = hit_v[pl.ds(0, _L)][0]
        hit_v[pl.ds(nh, _L)] = jnp.full((_L,), h0, jnp.int32)
        occv = jnp.full((_L,), occ, jnp.int32)

        def hit_block(b, carry):
            pv = hit_v[pl.ds(b * _L, _L)]
            offs = (pv >> 14) & (_CH - 1)
            slots = pv & (_BATCH - 1)
            below = occv & ((1 << (offs >> 7)) - 1)
            rank = (below & 1) + ((below >> 1) & 1) + ((below >> 2) & 1)
            offs = (rank << 7) | (offs & 127)
            par = (b & 1) * _L
            for lane in range(_L):
                off = offs[lane]
                slot = slots[lane]
                ring = par + lane
                offv = jnp.full((_L,), off, jnp.int32)
                for k in range(_EMBED // _L):
                    v = gather_row(iota + k * _L, offv)
                    ring_v[ring, pl.ds(k * _L, _L)] = v
                pltpu.async_copy(ring_v.at[pl.ds(ring, 1), :],
                                 out_hbm.at[pl.ds(slot, 1), :], sem_row)
            pltpu.make_async_copy(out_hbm.at[pl.ds(0, _L), :],
                                  ring_v.at[pl.ds(0, _L), :], sem_row).wait()
            return carry

        lax.fori_loop(0, (nh + _L - 1) >> 4, hit_block, 0)

    def chunk_body(g, carry):
        buf = g & 1
        full = g < 2
        m, k = group_count(g)
        wait_units(jnp.where(full, _GPC, k))
        mi = m.astype(jnp.int32)
        occ = jnp.where(
            full,
            (1 << _GPC) - 1,
            mi[0] + (mi[1] << 1) + (mi[2] << 2) + (mi[3] << 3),
        )
        bufv = jnp.full((_L,), buf, jnp.int32)
        process_chunk(g, lambda cv, ov: plsc.load_gather(slab_v, [bufv, cv, ov]),
                      occ)
        fetch_skip(jnp.minimum(g + 2, n_g - 1), buf)
        return carry

    lax.fori_loop(0, n_g, chunk_body, 0)
    _, k_last = group_count(n_g - 1)
    wait_units(2 * k_last)

    pltpu.async_copy(tail_hbm, tail_v, sem_slab)
    pltpu.make_async_copy(tail_hbm, tail_v, sem_slab).wait()
    process_chunk(_GFULL + (wid < _NEXTRA).astype(jnp.int32),
                  lambda cv, ov: plsc.load_gather(tail_v, [cv, ov]),
                  (1 << _GPC) - 1)


@jax.jit
def _embed_lookup(node_ids, node_embed_weight):
    run = pl.kernel(
        _gather_body,
        out_type=jax.ShapeDtypeStruct((_BATCH, _EMBED), jnp.float32),
        mesh=plsc.VectorSubcoreMesh(core_axis_name="c", subcore_axis_name="s"),
        scratch_types=[
            pltpu.VMEM((_BATCH,), jnp.int32),
            pltpu.VMEM((_BATCH + _L,), jnp.int32),
            pltpu.VMEM((_BATCH + _L,), jnp.int32),
            pltpu.VMEM((2, _EMBED, _CH), jnp.float32),
            pltpu.VMEM((_EMBED, _TAIL_LEN), jnp.float32),
            pltpu.VMEM((2 * _L, _EMBED), jnp.float32),
            pltpu.VMEM((_CNT_CAP,), jnp.int32),
            pltpu.VMEM((_L,), jnp.int32),
            pltpu.SemaphoreType.DMA,
            pltpu.SemaphoreType.DMA,
        ],
        compiler_params=pltpu.CompilerParams(needs_layout_passes=False),
    )
    tablet = node_embed_weight.T
    return run(node_ids, tablet, tablet[:, _TAIL_BASE:])


def kernel(node_ids, node_embed_weight):
    return _embed_lookup(node_ids.astype(jnp.int32), node_embed_weight)
